# baseline (speedup 1.0000x reference)
import functools

import jax
import jax.numpy as jnp
from jax import lax
from jax.experimental import pallas as pl
from jax.experimental.pallas import tpu as pltpu
from jax.experimental.pallas import tpu_sc as plsc

_VOCAB = 1000000
_DIM = 64
_B = 16384
_NEG = 20

_NC = 2
_NS = 16
_NW = _NC * _NS
_BW = _B // _NW
_C = 32
_NCHUNK = _BW // _C
_NEG_GATHER = 128
_NEG_STEPS = (_C * _NEG) // _NEG_GATHER


def _sc_body(cen_idx, ctx_idx, neg_idx, emb, oemb, scores_out,
             idx_cen_v, idx_ctx_v, idx_neg_v,
             cen_a, ctx_a, neg_a, cen_b, ctx_b, neg_b,
             part_v, scores_s, sem_a, sem_b):
    c = lax.axis_index("c")
    s = lax.axis_index("s")
    wid = s * _NC + c
    base = wid * _BW

    pltpu.sync_copy(cen_idx.at[pl.ds(base, _BW)], idx_cen_v)
    pltpu.sync_copy(ctx_idx.at[pl.ds(base, _BW)], idx_ctx_v)
    pltpu.sync_copy(neg_idx.at[pl.ds(base * _NEG, _BW * _NEG)], idx_neg_v)

    iota = jnp.arange(16, dtype=jnp.int32)

    def fire(ci, cen_v, ctx_v, neg_v, sem):
        cb = ci * _C
        pltpu.async_copy(emb.at[idx_cen_v.at[pl.ds(cb, _C)]], cen_v, sem)
        pltpu.async_copy(oemb.at[idx_ctx_v.at[pl.ds(cb, _C)]], ctx_v, sem)
        for j in range(_NEG_STEPS):
            pltpu.async_copy(
                oemb.at[idx_neg_v.at[pl.ds(cb * _NEG + j * _NEG_GATHER,
                                           _NEG_GATHER)]],
                neg_v.at[pl.ds(j * _NEG_GATHER, _NEG_GATHER)], sem)

    def drain(cen_v, ctx_v, neg_v, sem):
        pltpu.make_async_copy(emb.at[pl.ds(0, _C)], cen_v, sem).wait()
        pltpu.make_async_copy(oemb.at[pl.ds(0, _C)], ctx_v, sem).wait()
        for j in range(_NEG_STEPS):
            pltpu.make_async_copy(
                oemb.at[pl.ds(0, _NEG_GATHER)],
                neg_v.at[pl.ds(j * _NEG_GATHER, _NEG_GATHER)], sem).wait()

    def compute(ci, cen_v, ctx_v, neg_v):
        cb = ci * _C

        def b_body(b, carry2):
            pb = (b & 3) << 9
            c0 = cen_v[b, pl.ds(0, 16)]
            c1 = cen_v[b, pl.ds(16, 16)]
            c2 = cen_v[b, pl.ds(32, 16)]
            c3 = cen_v[b, pl.ds(48, 16)]
            for k in range(_NEG):
                r = b * _NEG + k
                t = (c0 * neg_v[r, pl.ds(0, 16)]
                     + c1 * neg_v[r, pl.ds(16, 16)]
                     + c2 * neg_v[r, pl.ds(32, 16)]
                     + c3 * neg_v[r, pl.ds(48, 16)])
                part_v[pl.ds(pb + k * 16, 16)] = plsc.cumsum(t)
            p = (c0 * ctx_v[b, pl.ds(0, 16)]
                 + c1 * ctx_v[b, pl.ds(16, 16)]
                 + c2 * ctx_v[b, pl.ds(32, 16)]
                 + c3 * ctx_v[b, pl.ds(48, 16)])
            part_v[pl.ds(pb + _NEG * 16, 16)] = plsc.cumsum(p)
            g1 = plsc.load_gather(part_v, [pb + iota * 16 + 15])
            g2 = plsc.load_gather(part_v, [pb + iota * 16 + 271])
            bb = cb + b
            scores_s[bb, pl.ds(0, 16)] = g1
            scores_s[bb, pl.ds(16, 16)] = g2
            return carry2

        lax.fori_loop(0, _C, b_body, 0, unroll=4)

    fire(0, cen_a, ctx_a, neg_a, sem_a)

    def super_body(h, carry):
        ci = h * 2
        fire(ci + 1, cen_b, ctx_b, neg_b, sem_b)
        drain(cen_a, ctx_a, neg_a, sem_a)
        compute(ci, cen_a, ctx_a, neg_a)

        @pl.when(ci + 2 < _NCHUNK)
        def _():
            fire(ci + 2, cen_a, ctx_a, neg_a, sem_a)

        drain(cen_b, ctx_b, neg_b, sem_b)
        compute(ci + 1, cen_b, ctx_b, neg_b)
        return carry

    lax.fori_loop(0, _NCHUNK // 2, super_body, 0, unroll=False)

    pltpu.sync_copy(scores_s, scores_out.at[pl.ds(base, _BW)])


@jax.jit
def _sc_call(cen_idx, ctx_idx, neg_idx, emb, oemb):
    mesh = plsc.VectorSubcoreMesh(core_axis_name="c", subcore_axis_name="s")
    return pl.kernel(
        _sc_body,
        out_type=jax.ShapeDtypeStruct((_B, 32), jnp.float32),
        mesh=mesh,
        scratch_types=[
            pltpu.VMEM((_BW,), jnp.int32),
            pltpu.VMEM((_BW,), jnp.int32),
            pltpu.VMEM((_BW * _NEG,), jnp.int32),
            pltpu.VMEM((_C, _DIM), jnp.float32),
            pltpu.VMEM((_C, _DIM), jnp.float32),
            pltpu.VMEM((_C * _NEG, _DIM), jnp.float32),
            pltpu.VMEM((_C, _DIM), jnp.float32),
            pltpu.VMEM((_C, _DIM), jnp.float32),
            pltpu.VMEM((_C * _NEG, _DIM), jnp.float32),
            pltpu.VMEM((2048,), jnp.float32),
            pltpu.VMEM((_BW, 32), jnp.float32),
            pltpu.SemaphoreType.DMA,
            pltpu.SemaphoreType.DMA,
        ],
        compiler_params=pltpu.CompilerParams(
            needs_layout_passes=False, use_tc_tiling_on_sc=False),
    )(cen_idx, ctx_idx, neg_idx, emb, oemb)


def kernel(center, context, negatives, embedding, output_embedding):
    cen = center.astype(jnp.int32)
    ctx = context.astype(jnp.int32)
    neg = negatives.astype(jnp.int32).reshape(-1)
    scores = _sc_call(cen, ctx, neg, embedding, output_embedding)
    return scores[:, 20], scores[:, :20]


# speedup vs baseline: 1.0026x; 1.0026x over previous
import functools

import jax
import jax.numpy as jnp
from jax import lax
from jax.experimental import pallas as pl
from jax.experimental.pallas import tpu as pltpu
from jax.experimental.pallas import tpu_sc as plsc

_VOCAB = 1000000
_DIM = 64
_B = 16384
_NEG = 20

_NC = 2
_NS = 16
_NW = _NC * _NS
_BW = _B // _NW
_C = 32
_NCHUNK = _BW // _C
_NEG_GATHER = 128
_NEG_STEPS = (_C * _NEG) // _NEG_GATHER


def _sc_body(cen_idx, ctx_idx, neg_idx, emb, oemb, scores_out,
             idx_cen_v, idx_ctx_v, idx_neg_v,
             cen_a, ctx_a, neg_a, cen_b, ctx_b, neg_b,
             part_v, scores_s, sem_a, sem_b):
    c = lax.axis_index("c")
    s = lax.axis_index("s")
    wid = s * _NC + c
    base = wid * _BW

    pltpu.sync_copy(cen_idx.at[pl.ds(base, _BW)], idx_cen_v)
    pltpu.sync_copy(ctx_idx.at[pl.ds(base, _BW)], idx_ctx_v)
    pltpu.sync_copy(neg_idx.at[pl.ds(base * _NEG, _BW * _NEG)], idx_neg_v)

    iota = jnp.arange(16, dtype=jnp.int32)

    def fire(ci, cen_v, ctx_v, neg_v, sem):
        cb = ci * _C
        pltpu.async_copy(emb.at[idx_cen_v.at[pl.ds(cb, _C)]], cen_v, sem)
        pltpu.async_copy(oemb.at[idx_ctx_v.at[pl.ds(cb, _C)]], ctx_v, sem)
        for j in range(_NEG_STEPS):
            pltpu.async_copy(
                oemb.at[idx_neg_v.at[pl.ds(cb * _NEG + j * _NEG_GATHER,
                                           _NEG_GATHER)]],
                neg_v.at[pl.ds(j * _NEG_GATHER, _NEG_GATHER)], sem)

    def drain(cen_v, ctx_v, neg_v, sem):
        pltpu.make_async_copy(emb.at[pl.ds(0, _C)], cen_v, sem).wait()
        pltpu.make_async_copy(oemb.at[pl.ds(0, _C)], ctx_v, sem).wait()
        for j in range(_NEG_STEPS):
            pltpu.make_async_copy(
                oemb.at[pl.ds(0, _NEG_GATHER)],
                neg_v.at[pl.ds(j * _NEG_GATHER, _NEG_GATHER)], sem).wait()

    def compute(ci, cen_v, ctx_v, neg_v):
        cb = ci * _C

        def b_body(b, carry2):
            pb = (b & 1) << 9
            c0 = cen_v[b, pl.ds(0, 16)]
            c1 = cen_v[b, pl.ds(16, 16)]
            c2 = cen_v[b, pl.ds(32, 16)]
            c3 = cen_v[b, pl.ds(48, 16)]
            for k in range(_NEG):
                r = b * _NEG + k
                t = (c0 * neg_v[r, pl.ds(0, 16)]
                     + c1 * neg_v[r, pl.ds(16, 16)]
                     + c2 * neg_v[r, pl.ds(32, 16)]
                     + c3 * neg_v[r, pl.ds(48, 16)])
                part_v[pl.ds(pb + k * 16, 16)] = plsc.cumsum(t)
            p = (c0 * ctx_v[b, pl.ds(0, 16)]
                 + c1 * ctx_v[b, pl.ds(16, 16)]
                 + c2 * ctx_v[b, pl.ds(32, 16)]
                 + c3 * ctx_v[b, pl.ds(48, 16)])
            part_v[pl.ds(pb + _NEG * 16, 16)] = plsc.cumsum(p)
            g1 = plsc.load_gather(part_v, [pb + iota * 16 + 15])
            g2 = plsc.load_gather(part_v, [pb + iota * 16 + 271])
            bb = cb + b
            scores_s[bb, pl.ds(0, 16)] = g1
            scores_s[bb, pl.ds(16, 16)] = g2
            return carry2

        lax.fori_loop(0, _C, b_body, 0, unroll=2)

    fire(0, cen_a, ctx_a, neg_a, sem_a)

    def super_body(h, carry):
        ci = h * 2
        fire(ci + 1, cen_b, ctx_b, neg_b, sem_b)
        drain(cen_a, ctx_a, neg_a, sem_a)
        compute(ci, cen_a, ctx_a, neg_a)

        @pl.when(ci + 2 < _NCHUNK)
        def _():
            fire(ci + 2, cen_a, ctx_a, neg_a, sem_a)

        drain(cen_b, ctx_b, neg_b, sem_b)
        compute(ci + 1, cen_b, ctx_b, neg_b)
        return carry

    lax.fori_loop(0, _NCHUNK // 2, super_body, 0, unroll=False)

    pltpu.sync_copy(scores_s, scores_out.at[pl.ds(base, _BW)])


@jax.jit
def _sc_call(cen_idx, ctx_idx, neg_idx, emb, oemb):
    mesh = plsc.VectorSubcoreMesh(core_axis_name="c", subcore_axis_name="s")
    return pl.kernel(
        _sc_body,
        out_type=jax.ShapeDtypeStruct((_B, 32), jnp.float32),
        mesh=mesh,
        scratch_types=[
            pltpu.VMEM((_BW,), jnp.int32),
            pltpu.VMEM((_BW,), jnp.int32),
            pltpu.VMEM((_BW * _NEG,), jnp.int32),
            pltpu.VMEM((_C, _DIM), jnp.float32),
            pltpu.VMEM((_C, _DIM), jnp.float32),
            pltpu.VMEM((_C * _NEG, _DIM), jnp.float32),
            pltpu.VMEM((_C, _DIM), jnp.float32),
            pltpu.VMEM((_C, _DIM), jnp.float32),
            pltpu.VMEM((_C * _NEG, _DIM), jnp.float32),
            pltpu.VMEM((1024,), jnp.float32),
            pltpu.VMEM((_BW, 32), jnp.float32),
            pltpu.SemaphoreType.DMA,
            pltpu.SemaphoreType.DMA,
        ],
        compiler_params=pltpu.CompilerParams(
            needs_layout_passes=False, use_tc_tiling_on_sc=False),
    )(cen_idx, ctx_idx, neg_idx, emb, oemb)


def kernel(center, context, negatives, embedding, output_embedding):
    cen = center.astype(jnp.int32)
    ctx = context.astype(jnp.int32)
    neg = negatives.astype(jnp.int32).reshape(-1)
    scores = _sc_call(cen, ctx, neg, embedding, output_embedding)
    return scores[:, 20], scores[:, :20]
